# Initial kernel scaffold; baseline (speedup 1.0000x reference)
import functools

import jax
import jax.numpy as jnp
from jax import lax
from jax.experimental import pallas as pl
from jax.experimental.pallas import tpu as pltpu
from jax.experimental.pallas import tpu_sc as plsc

D_MODEL = 128
LANES = 16
NUM_CORES = 2
NUM_SUBCORES = 16
NUM_WORKERS = NUM_CORES * NUM_SUBCORES

B_TOTAL = 4096 * 200
ROWS_PER_WORKER = B_TOTAL // NUM_WORKERS
CHUNK = 512
NUM_CHUNKS = ROWS_PER_WORKER // CHUNK

_SCALE = jnp.float32(D_MODEL ** 0.5)


def _sc_body(x_hbm, table_hbm, out_hbm, idx_v, rows_v, gsem):
    wid = lax.axis_index("s") * NUM_CORES + lax.axis_index("c")
    base = wid * ROWS_PER_WORKER

    def chunk_body(g, carry):
        cbase = base + g * CHUNK
        pltpu.sync_copy(x_hbm.at[pl.ds(cbase, CHUNK)], idx_v)
        pltpu.async_copy(table_hbm.at[idx_v], rows_v, gsem).wait()

        def row_body(r, c):
            for j in range(D_MODEL // LANES):
                sl = pl.ds(j * LANES, LANES)
                rows_v[r, sl] = rows_v[r, sl] * _SCALE
            return c

        lax.fori_loop(0, CHUNK, row_body, 0)
        pltpu.sync_copy(rows_v, out_hbm.at[pl.ds(cbase, CHUNK)])
        return carry

    lax.fori_loop(0, NUM_CHUNKS, chunk_body, 0)


@jax.jit
def kernel(x, embedding):
    idx = x.reshape(B_TOTAL)
    mesh = plsc.VectorSubcoreMesh(core_axis_name="c", subcore_axis_name="s")
    out = pl.kernel(
        _sc_body,
        out_type=jax.ShapeDtypeStruct((B_TOTAL, D_MODEL), jnp.float32),
        mesh=mesh,
        scratch_types=[
            pltpu.VMEM((CHUNK,), jnp.int32),
            pltpu.VMEM((CHUNK, D_MODEL), jnp.float32),
            pltpu.SemaphoreType.DMA,
        ],
    )(idx, embedding)
    return out.reshape(x.shape + (D_MODEL,))


# speedup vs baseline: 1.2450x; 1.2450x over previous
import functools

import jax
import jax.numpy as jnp
from jax import lax
from jax.experimental import pallas as pl
from jax.experimental.pallas import tpu as pltpu
from jax.experimental.pallas import tpu_sc as plsc

D_MODEL = 128
LANES = 16
NUM_CORES = 2
NUM_SUBCORES = 16
NUM_WORKERS = NUM_CORES * NUM_SUBCORES

B_TOTAL = 4096 * 200
ROWS_PER_WORKER = B_TOTAL // NUM_WORKERS
CHUNK = 512
NUM_CHUNKS = ROWS_PER_WORKER // CHUNK

_SCALE = float(D_MODEL ** 0.5)


def _sc_body(x_hbm, table_hbm, out_hbm, idx_v, rows_v, gsem):
    wid = lax.axis_index("s") * NUM_CORES + lax.axis_index("c")
    base = wid * ROWS_PER_WORKER

    def chunk_body(g, carry):
        cbase = base + g * CHUNK
        pltpu.sync_copy(x_hbm.at[pl.ds(cbase, CHUNK)], idx_v)
        pltpu.async_copy(table_hbm.at[idx_v], rows_v, gsem).wait()

        def row_body(r, c):
            for j in range(D_MODEL // LANES):
                sl = pl.ds(j * LANES, LANES)
                rows_v[r, sl] = rows_v[r, sl] * _SCALE
            return c

        lax.fori_loop(0, CHUNK, row_body, 0)
        pltpu.sync_copy(rows_v, out_hbm.at[pl.ds(cbase, CHUNK)])
        return carry

    lax.fori_loop(0, NUM_CHUNKS, chunk_body, 0)


@jax.jit
def kernel(x, embedding):
    idx = x.reshape(B_TOTAL)
    mesh = plsc.VectorSubcoreMesh(core_axis_name="c", subcore_axis_name="s")
    out = pl.kernel(
        _sc_body,
        out_type=jax.ShapeDtypeStruct((B_TOTAL, D_MODEL), jnp.float32),
        mesh=mesh,
        scratch_types=[
            pltpu.VMEM((CHUNK,), jnp.int32),
            pltpu.VMEM((CHUNK, D_MODEL), jnp.float32),
            pltpu.SemaphoreType.DMA,
        ],
    )(idx, embedding)
    return out.reshape(x.shape + (D_MODEL,))


# speedup vs baseline: 1.8203x; 1.4620x over previous
import jax
import jax.numpy as jnp
from jax import lax
from jax.experimental import pallas as pl
from jax.experimental.pallas import tpu as pltpu
from jax.experimental.pallas import tpu_sc as plsc

D_MODEL = 128
LANES = 16
NUM_CORES = 2
NUM_SUBCORES = 16
NUM_WORKERS = NUM_CORES * NUM_SUBCORES

B_TOTAL = 4096 * 200
ROWS_PER_WORKER = B_TOTAL // NUM_WORKERS
CHUNK = 400
NUM_CHUNKS = ROWS_PER_WORKER // CHUNK

_SCALE = float(D_MODEL ** 0.5)


def _sc_body(x_hbm, table_hbm, out_hbm,
             idx0, idx1, rows0, rows1, gsem0, gsem1):
    idx_v = (idx0, idx1)
    rows_v = (rows0, rows1)
    gsem = (gsem0, gsem1)

    wid = lax.axis_index("s") * NUM_CORES + lax.axis_index("c")
    base = wid * ROWS_PER_WORKER

    def start_gather(b, g):
        cbase = base + g * CHUNK
        pltpu.sync_copy(x_hbm.at[pl.ds(cbase, CHUNK)], idx_v[b])
        pltpu.async_copy(table_hbm.at[idx_v[b]], rows_v[b], gsem[b])

    def wait_gather(b):
        pltpu.make_async_copy(table_hbm.at[idx_v[b]], rows_v[b], gsem[b]).wait()

    def scale_rows(b):
        def row_body(r, c):
            for u in range(2):
                for j in range(D_MODEL // LANES):
                    sl = pl.ds(j * LANES, LANES)
                    rows_v[b][2 * r + u, sl] = rows_v[b][2 * r + u, sl] * _SCALE
            return c
        lax.fori_loop(0, CHUNK // 2, row_body, 0)

    start_gather(0, 0)

    def outer(gg, carry):
        for b in range(2):
            g = 2 * gg + b
            wait_gather(b)

            @pl.when(g + 1 < NUM_CHUNKS)
            def _():
                start_gather(1 - b, g + 1)

            scale_rows(b)
            pltpu.sync_copy(rows_v[b], out_hbm.at[pl.ds(base + g * CHUNK, CHUNK)])
        return carry

    lax.fori_loop(0, NUM_CHUNKS // 2, outer, 0)


@jax.jit
def kernel(x, embedding):
    idx = x.reshape(B_TOTAL)
    mesh = plsc.VectorSubcoreMesh(core_axis_name="c", subcore_axis_name="s")
    out = pl.kernel(
        _sc_body,
        out_type=jax.ShapeDtypeStruct((B_TOTAL, D_MODEL), jnp.float32),
        mesh=mesh,
        scratch_types=[
            pltpu.VMEM((CHUNK,), jnp.int32),
            pltpu.VMEM((CHUNK,), jnp.int32),
            pltpu.VMEM((CHUNK, D_MODEL), jnp.float32),
            pltpu.VMEM((CHUNK, D_MODEL), jnp.float32),
            pltpu.SemaphoreType.DMA,
            pltpu.SemaphoreType.DMA,
        ],
    )(idx, embedding)
    return out.reshape(x.shape + (D_MODEL,))


# speedup vs baseline: 1.8203x; 1.0000x over previous
import jax
import jax.numpy as jnp
from jax import lax
from jax.experimental import pallas as pl
from jax.experimental.pallas import tpu as pltpu
from jax.experimental.pallas import tpu_sc as plsc

D_MODEL = 128
LANES = 16
NUM_CORES = 2
NUM_SUBCORES = 16
NUM_WORKERS = NUM_CORES * NUM_SUBCORES

B_TOTAL = 4096 * 200
ROWS_PER_WORKER = B_TOTAL // NUM_WORKERS
CHUNK = 400
NUM_CHUNKS = ROWS_PER_WORKER // CHUNK

_SCALE = float(D_MODEL ** 0.5)


def _sc_body(x_hbm, table_hbm, out_hbm,
             idx0, idx1, rows0, rows1, gsem0, gsem1, ssem0, ssem1):
    idx_v = (idx0, idx1)
    rows_v = (rows0, rows1)
    gsem = (gsem0, gsem1)
    ssem = (ssem0, ssem1)

    wid = lax.axis_index("s") * NUM_CORES + lax.axis_index("c")
    base = wid * ROWS_PER_WORKER

    def start_gather(b, g):
        cbase = base + g * CHUNK
        pltpu.sync_copy(x_hbm.at[pl.ds(cbase, CHUNK)], idx_v[b])
        pltpu.async_copy(table_hbm.at[idx_v[b]], rows_v[b], gsem[b])

    def wait_gather(b):
        pltpu.make_async_copy(table_hbm.at[idx_v[b]], rows_v[b], gsem[b]).wait()

    def start_scatter(b, g):
        pltpu.async_copy(rows_v[b], out_hbm.at[pl.ds(base + g * CHUNK, CHUNK)],
                         ssem[b])

    def wait_scatter(b, g):
        pltpu.make_async_copy(rows_v[b],
                              out_hbm.at[pl.ds(base + g * CHUNK, CHUNK)],
                              ssem[b]).wait()

    def scale_rows(b):
        def row_body(r, c):
            for u in range(2):
                for j in range(D_MODEL // LANES):
                    sl = pl.ds(j * LANES, LANES)
                    rows_v[b][2 * r + u, sl] = rows_v[b][2 * r + u, sl] * _SCALE
            return c
        lax.fori_loop(0, CHUNK // 2, row_body, 0)

    start_gather(0, 0)

    def outer(gg, carry):
        for b in range(2):
            g = 2 * gg + b
            wait_gather(b)

            @pl.when(g + 1 < NUM_CHUNKS)
            def _():
                @pl.when(g >= 1)
                def _():
                    wait_scatter(1 - b, g - 1)

                start_gather(1 - b, g + 1)

            scale_rows(b)
            start_scatter(b, g)
        return carry

    lax.fori_loop(0, NUM_CHUNKS // 2, outer, 0)
    wait_scatter(0, NUM_CHUNKS - 2)
    wait_scatter(1, NUM_CHUNKS - 1)


@jax.jit
def kernel(x, embedding):
    idx = x.reshape(B_TOTAL)
    mesh = plsc.VectorSubcoreMesh(core_axis_name="c", subcore_axis_name="s")
    out = pl.kernel(
        _sc_body,
        out_type=jax.ShapeDtypeStruct((B_TOTAL, D_MODEL), jnp.float32),
        mesh=mesh,
        scratch_types=[
            pltpu.VMEM((CHUNK,), jnp.int32),
            pltpu.VMEM((CHUNK,), jnp.int32),
            pltpu.VMEM((CHUNK, D_MODEL), jnp.float32),
            pltpu.VMEM((CHUNK, D_MODEL), jnp.float32),
            pltpu.SemaphoreType.DMA,
            pltpu.SemaphoreType.DMA,
            pltpu.SemaphoreType.DMA,
            pltpu.SemaphoreType.DMA,
        ],
    )(idx, embedding)
    return out.reshape(x.shape + (D_MODEL,))


# speedup vs baseline: 1.8244x; 1.0022x over previous
import jax
import jax.numpy as jnp
from jax import lax
from jax.experimental import pallas as pl
from jax.experimental.pallas import tpu as pltpu
from jax.experimental.pallas import tpu_sc as plsc

D_MODEL = 128
LANES = 16
NUM_CORES = 2
NUM_SUBCORES = 16
NUM_WORKERS = NUM_CORES * NUM_SUBCORES

B_TOTAL = 4096 * 200
ROWS_PER_WORKER = B_TOTAL // NUM_WORKERS
CHUNK = 400
NUM_CHUNKS = ROWS_PER_WORKER // CHUNK

_SCALE = float(D_MODEL ** 0.5)


def _sc_body(x_hbm, table_hbm, out_hbm,
             idx_all, rows0, rows1, gsem0, gsem1, ssem0, ssem1):
    rows_v = (rows0, rows1)
    gsem = (gsem0, gsem1)
    ssem = (ssem0, ssem1)

    wid = lax.axis_index("s") * NUM_CORES + lax.axis_index("c")
    base = wid * ROWS_PER_WORKER

    pltpu.sync_copy(x_hbm.at[pl.ds(base, ROWS_PER_WORKER)], idx_all)

    def idx_slice(g):
        return idx_all.at[pl.ds(g * CHUNK, CHUNK)]

    def start_gather(b, g):
        pltpu.async_copy(table_hbm.at[idx_slice(g)], rows_v[b], gsem[b])

    def wait_gather(b, g):
        pltpu.make_async_copy(table_hbm.at[idx_slice(g)], rows_v[b],
                              gsem[b]).wait()

    def start_scatter(b, g):
        pltpu.async_copy(rows_v[b], out_hbm.at[pl.ds(base + g * CHUNK, CHUNK)],
                         ssem[b])

    def wait_scatter(b, g):
        pltpu.make_async_copy(rows_v[b],
                              out_hbm.at[pl.ds(base + g * CHUNK, CHUNK)],
                              ssem[b]).wait()

    def scale_rows(b):
        def row_body(r, c):
            for u in range(2):
                for j in range(D_MODEL // LANES):
                    sl = pl.ds(j * LANES, LANES)
                    rows_v[b][2 * r + u, sl] = rows_v[b][2 * r + u, sl] * _SCALE
            return c
        lax.fori_loop(0, CHUNK // 2, row_body, 0)

    start_gather(0, 0)

    def outer(gg, carry):
        for b in range(2):
            g = 2 * gg + b
            wait_gather(b, g)

            @pl.when(g + 1 < NUM_CHUNKS)
            def _():
                @pl.when(g >= 1)
                def _():
                    wait_scatter(1 - b, g - 1)

                start_gather(1 - b, g + 1)

            scale_rows(b)
            start_scatter(b, g)
        return carry

    lax.fori_loop(0, NUM_CHUNKS // 2, outer, 0)
    wait_scatter(0, NUM_CHUNKS - 2)
    wait_scatter(1, NUM_CHUNKS - 1)


@jax.jit
def kernel(x, embedding):
    idx = x.reshape(B_TOTAL)
    mesh = plsc.VectorSubcoreMesh(core_axis_name="c", subcore_axis_name="s")
    out = pl.kernel(
        _sc_body,
        out_type=jax.ShapeDtypeStruct((B_TOTAL, D_MODEL), jnp.float32),
        mesh=mesh,
        scratch_types=[
            pltpu.VMEM((ROWS_PER_WORKER,), jnp.int32),
            pltpu.VMEM((CHUNK, D_MODEL), jnp.float32),
            pltpu.VMEM((CHUNK, D_MODEL), jnp.float32),
            pltpu.SemaphoreType.DMA,
            pltpu.SemaphoreType.DMA,
            pltpu.SemaphoreType.DMA,
            pltpu.SemaphoreType.DMA,
        ],
    )(idx, embedding)
    return out.reshape(x.shape + (D_MODEL,))


# speedup vs baseline: 1.8467x; 1.0122x over previous
import jax
import jax.numpy as jnp
from jax import lax
from jax.experimental import pallas as pl
from jax.experimental.pallas import tpu as pltpu
from jax.experimental.pallas import tpu_sc as plsc

D_MODEL = 128
LANES = 16
NUM_CORES = 2
NUM_SUBCORES = 16
NUM_WORKERS = NUM_CORES * NUM_SUBCORES

B_TOTAL = 4096 * 200
ROWS_PER_WORKER = B_TOTAL // NUM_WORKERS
CHUNK = 200
NUM_CHUNKS = ROWS_PER_WORKER // CHUNK
NBUF = 4

_SCALE = float(D_MODEL ** 0.5)


def _sc_body(x_hbm, table_hbm, out_hbm, idx_all,
             rows0, rows1, rows2, rows3,
             gsem0, gsem1, gsem2, gsem3,
             ssem0, ssem1, ssem2, ssem3):
    rows_v = (rows0, rows1, rows2, rows3)
    gsem = (gsem0, gsem1, gsem2, gsem3)
    ssem = (ssem0, ssem1, ssem2, ssem3)

    wid = lax.axis_index("s") * NUM_CORES + lax.axis_index("c")
    base = wid * ROWS_PER_WORKER

    pltpu.sync_copy(x_hbm.at[pl.ds(base, ROWS_PER_WORKER)], idx_all)

    def idx_slice(g):
        return idx_all.at[pl.ds(g * CHUNK, CHUNK)]

    def start_gather(b, g):
        pltpu.async_copy(table_hbm.at[idx_slice(g)], rows_v[b], gsem[b])

    def wait_gather(b, g):
        pltpu.make_async_copy(table_hbm.at[idx_slice(g)], rows_v[b],
                              gsem[b]).wait()

    def start_scatter(b, g):
        pltpu.async_copy(rows_v[b], out_hbm.at[pl.ds(base + g * CHUNK, CHUNK)],
                         ssem[b])

    def wait_scatter(b, g):
        pltpu.make_async_copy(rows_v[b],
                              out_hbm.at[pl.ds(base + g * CHUNK, CHUNK)],
                              ssem[b]).wait()

    def scale_rows(b):
        def row_body(r, c):
            for u in range(2):
                for j in range(D_MODEL // LANES):
                    sl = pl.ds(j * LANES, LANES)
                    rows_v[b][2 * r + u, sl] = rows_v[b][2 * r + u, sl] * _SCALE
            return c
        lax.fori_loop(0, CHUNK // 2, row_body, 0)

    start_gather(0, 0)
    start_gather(1, 1)

    def outer(gg, carry):
        for b in range(NBUF):
            g = NBUF * gg + b
            wait_gather(b, g)

            @pl.when(g + 2 < NUM_CHUNKS)
            def _():
                bn = (b + 2) % NBUF
                @pl.when(g >= 2)
                def _():
                    wait_scatter(bn, g - 2)

                start_gather(bn, g + 2)

            scale_rows(b)
            start_scatter(b, g)
        return carry

    lax.fori_loop(0, NUM_CHUNKS // NBUF, outer, 0)
    wait_scatter(2, NUM_CHUNKS - 2)
    wait_scatter(3, NUM_CHUNKS - 1)


@jax.jit
def kernel(x, embedding):
    idx = x.reshape(B_TOTAL)
    mesh = plsc.VectorSubcoreMesh(core_axis_name="c", subcore_axis_name="s")
    out = pl.kernel(
        _sc_body,
        out_type=jax.ShapeDtypeStruct((B_TOTAL, D_MODEL), jnp.float32),
        mesh=mesh,
        scratch_types=(
            [pltpu.VMEM((ROWS_PER_WORKER,), jnp.int32)]
            + [pltpu.VMEM((CHUNK, D_MODEL), jnp.float32)] * NBUF
            + [pltpu.SemaphoreType.DMA] * (2 * NBUF)
        ),
    )(idx, embedding)
    return out.reshape(x.shape + (D_MODEL,))


# speedup vs baseline: 1.8520x; 1.0029x over previous
import jax
import jax.numpy as jnp
from jax import lax
from jax.experimental import pallas as pl
from jax.experimental.pallas import tpu as pltpu
from jax.experimental.pallas import tpu_sc as plsc

D_MODEL = 128
LANES = 16
NUM_CORES = 2
NUM_SUBCORES = 16
NUM_WORKERS = NUM_CORES * NUM_SUBCORES

B_TOTAL = 4096 * 200
ROWS_PER_WORKER = B_TOTAL // NUM_WORKERS
CHUNK = 128
NUM_CHUNKS = ROWS_PER_WORKER // CHUNK
NBUF = 4

_SCALE = float(D_MODEL ** 0.5)


def _sc_body(x_hbm, table_hbm, out_hbm, idx_all,
             rows0, rows1, rows2, rows3,
             gsem0, gsem1, gsem2, gsem3,
             ssem0, ssem1, ssem2, ssem3):
    rows_v = (rows0, rows1, rows2, rows3)
    gsem = (gsem0, gsem1, gsem2, gsem3)
    ssem = (ssem0, ssem1, ssem2, ssem3)

    wid = lax.axis_index("s") * NUM_CORES + lax.axis_index("c")
    base = wid * ROWS_PER_WORKER

    pltpu.sync_copy(x_hbm.at[pl.ds(base, ROWS_PER_WORKER)], idx_all)

    def idx_slice(g):
        return idx_all.at[pl.ds(g * CHUNK, CHUNK)]

    def start_gather(b, g):
        pltpu.async_copy(table_hbm.at[idx_slice(g)], rows_v[b], gsem[b])

    def wait_gather(b, g):
        pltpu.make_async_copy(table_hbm.at[idx_slice(g)], rows_v[b],
                              gsem[b]).wait()

    def start_scatter(b, g):
        pltpu.async_copy(rows_v[b], out_hbm.at[pl.ds(base + g * CHUNK, CHUNK)],
                         ssem[b])

    def wait_scatter(b, g):
        pltpu.make_async_copy(rows_v[b],
                              out_hbm.at[pl.ds(base + g * CHUNK, CHUNK)],
                              ssem[b]).wait()

    def scale_rows(b):
        def row_body(r, c):
            for u in range(2):
                for j in range(D_MODEL // LANES):
                    sl = pl.ds(j * LANES, LANES)
                    rows_v[b][2 * r + u, sl] = rows_v[b][2 * r + u, sl] * _SCALE
            return c
        lax.fori_loop(0, CHUNK // 2, row_body, 0)

    start_gather(0, 0)
    start_gather(1, 1)

    def outer(gg, carry):
        for b in range(NBUF):
            g = NBUF * gg + b
            wait_gather(b, g)

            @pl.when(g + 2 < NUM_CHUNKS)
            def _():
                bn = (b + 2) % NBUF
                @pl.when(g >= 2)
                def _():
                    wait_scatter(bn, g - 2)

                start_gather(bn, g + 2)

            scale_rows(b)
            start_scatter(b, g)
        return carry

    lax.fori_loop(0, NUM_CHUNKS // NBUF, outer, 0)
    wait_scatter(2, NUM_CHUNKS - 2)
    wait_scatter(3, NUM_CHUNKS - 1)


@jax.jit
def kernel(x, embedding):
    idx = x.reshape(B_TOTAL)
    mesh = plsc.VectorSubcoreMesh(core_axis_name="c", subcore_axis_name="s")
    out = pl.kernel(
        _sc_body,
        out_type=jax.ShapeDtypeStruct((B_TOTAL, D_MODEL), jnp.float32),
        mesh=mesh,
        scratch_types=(
            [pltpu.VMEM((ROWS_PER_WORKER,), jnp.int32)]
            + [pltpu.VMEM((CHUNK, D_MODEL), jnp.float32)] * NBUF
            + [pltpu.SemaphoreType.DMA] * (2 * NBUF)
        ),
    )(idx, embedding)
    return out.reshape(x.shape + (D_MODEL,))
